# initial kernel scaffold (unmeasured)
import jax
import jax.numpy as jnp
from jax import lax
from jax.experimental import pallas as pl
from jax.experimental.pallas import tpu as pltpu

N_RING = 4
SEQ = 1024
HEADS = 16
HDIM = 128
QB = 256
SCALE = HDIM ** -0.5


def kernel(Q, K, V):
    q = Q[0].astype(jnp.bfloat16)
    k = K[0].astype(jnp.bfloat16)
    v = V[0].astype(jnp.bfloat16)

    def body(q_ref, k_ref, v_ref, out_ref, commk, commv, sendk, recvk, sendv, recvv):
        my_x = lax.axis_index("x")
        my_y = lax.axis_index("y")
        my_z = lax.axis_index("z")
        right = lax.rem(my_y + 1, N_RING)
        left = lax.rem(my_y + N_RING - 1, N_RING)

        barrier_sem = pltpu.get_barrier_semaphore()
        for nbr in (left, right):
            pl.semaphore_signal(
                barrier_sem, inc=1,
                device_id=(my_x, nbr, my_z),
                device_id_type=pl.DeviceIdType.MESH,
            )
        pl.semaphore_wait(barrier_sem, 2)

        for h in range(N_RING - 1):
            src_k = k_ref if h == 0 else commk.at[h - 1]
            src_v = v_ref if h == 0 else commv.at[h - 1]
            rdma_k = pltpu.make_async_remote_copy(
                src_ref=src_k, dst_ref=commk.at[h],
                send_sem=sendk.at[h], recv_sem=recvk.at[h],
                device_id=(my_x, right, my_z),
                device_id_type=pl.DeviceIdType.MESH,
            )
            rdma_v = pltpu.make_async_remote_copy(
                src_ref=src_v, dst_ref=commv.at[h],
                send_sem=sendv.at[h], recv_sem=recvv.at[h],
                device_id=(my_x, right, my_z),
                device_id_type=pl.DeviceIdType.MESH,
            )
            rdma_k.start()
            rdma_v.start()
            rdma_k.wait()
            rdma_v.wait()

        for hd in range(HEADS):
            k_full = jnp.concatenate(
                [k_ref[:, hd, :]] + [commk[h, :, hd, :] for h in range(N_RING - 1)],
                axis=0,
            )
            v_full = jnp.concatenate(
                [v_ref[:, hd, :]] + [commv[h, :, hd, :] for h in range(N_RING - 1)],
                axis=0,
            )
            for qb in range(0, SEQ, QB):
                q_blk = q_ref[qb:qb + QB, hd, :]
                s = lax.dot_general(
                    q_blk, k_full,
                    (((1,), (1,)), ((), ())),
                    preferred_element_type=jnp.float32,
                ) * SCALE
                m = jnp.max(s, axis=1, keepdims=True)
                e = jnp.exp(s - m)
                l = jnp.sum(e, axis=1, keepdims=True)
                o = jnp.dot(
                    e.astype(jnp.bfloat16), v_full,
                    preferred_element_type=jnp.float32,
                )
                out_ref[qb:qb + QB, hd, :] = o / l

    out = pl.pallas_call(
        body,
        out_shape=jax.ShapeDtypeStruct((SEQ, HEADS, HDIM), jnp.float32),
        in_specs=[pl.BlockSpec(memory_space=pltpu.VMEM)] * 3,
        out_specs=pl.BlockSpec(memory_space=pltpu.VMEM),
        scratch_shapes=[
            pltpu.VMEM((N_RING - 1, SEQ, HEADS, HDIM), jnp.bfloat16),
            pltpu.VMEM((N_RING - 1, SEQ, HEADS, HDIM), jnp.bfloat16),
            pltpu.SemaphoreType.DMA((N_RING - 1,)),
            pltpu.SemaphoreType.DMA((N_RING - 1,)),
            pltpu.SemaphoreType.DMA((N_RING - 1,)),
            pltpu.SemaphoreType.DMA((N_RING - 1,)),
        ],
        compiler_params=pltpu.CompilerParams(collective_id=0),
    )(q, k, v)
    return out[None]


# baseline (device time: 418993 ns/iter reference)
import jax
import jax.numpy as jnp
from jax import lax
from jax.experimental import pallas as pl
from jax.experimental.pallas import tpu as pltpu

N_RING = 4
SEQ = 1024
HEADS = 16
HDIM = 128
QB = 256
SCALE = HDIM ** -0.5


def kernel(Q, K, V):
    q = Q[0].transpose(1, 0, 2).astype(jnp.bfloat16)
    k = K[0].transpose(1, 0, 2).astype(jnp.bfloat16)
    v = V[0].transpose(1, 0, 2).astype(jnp.bfloat16)

    def body(q_ref, k_ref, v_ref, out_ref, commk, commv, sendk, recvk, sendv, recvv):
        my_x = lax.axis_index("x")
        my_y = lax.axis_index("y")
        my_z = lax.axis_index("z")
        right = lax.rem(my_y + 1, N_RING)
        left = lax.rem(my_y + N_RING - 1, N_RING)

        barrier_sem = pltpu.get_barrier_semaphore()
        for nbr in (left, right):
            pl.semaphore_signal(
                barrier_sem, inc=1,
                device_id=(my_x, nbr, my_z),
                device_id_type=pl.DeviceIdType.MESH,
            )
        pl.semaphore_wait(barrier_sem, 2)

        for h in range(N_RING - 1):
            src_k = k_ref if h == 0 else commk.at[h - 1]
            src_v = v_ref if h == 0 else commv.at[h - 1]
            rdma_k = pltpu.make_async_remote_copy(
                src_ref=src_k, dst_ref=commk.at[h],
                send_sem=sendk.at[h], recv_sem=recvk.at[h],
                device_id=(my_x, right, my_z),
                device_id_type=pl.DeviceIdType.MESH,
            )
            rdma_v = pltpu.make_async_remote_copy(
                src_ref=src_v, dst_ref=commv.at[h],
                send_sem=sendv.at[h], recv_sem=recvv.at[h],
                device_id=(my_x, right, my_z),
                device_id_type=pl.DeviceIdType.MESH,
            )
            rdma_k.start()
            rdma_v.start()
            rdma_k.wait()
            rdma_v.wait()

        def head_step(hd, _):
            k_full = jnp.concatenate(
                [k_ref[hd]] + [commk[h, hd] for h in range(N_RING - 1)],
                axis=0,
            )
            v_full = jnp.concatenate(
                [v_ref[hd]] + [commv[h, hd] for h in range(N_RING - 1)],
                axis=0,
            )
            for qb in range(0, SEQ, QB):
                q_blk = q_ref[hd, qb:qb + QB, :]
                s = lax.dot_general(
                    q_blk, k_full,
                    (((1,), (1,)), ((), ())),
                    preferred_element_type=jnp.float32,
                ) * SCALE
                m = jnp.max(s, axis=1, keepdims=True)
                e = jnp.exp(s - m)
                l = jnp.sum(e, axis=1, keepdims=True)
                o = jnp.dot(
                    e.astype(jnp.bfloat16), v_full,
                    preferred_element_type=jnp.float32,
                )
                out_ref[hd, qb:qb + QB, :] = o / l
            return _

        lax.fori_loop(0, HEADS, head_step, 0)

    out = pl.pallas_call(
        body,
        out_shape=jax.ShapeDtypeStruct((HEADS, SEQ, HDIM), jnp.float32),
        in_specs=[pl.BlockSpec(memory_space=pltpu.VMEM)] * 3,
        out_specs=pl.BlockSpec(memory_space=pltpu.VMEM),
        scratch_shapes=[
            pltpu.VMEM((N_RING - 1, HEADS, SEQ, HDIM), jnp.bfloat16),
            pltpu.VMEM((N_RING - 1, HEADS, SEQ, HDIM), jnp.bfloat16),
            pltpu.SemaphoreType.DMA((N_RING - 1,)),
            pltpu.SemaphoreType.DMA((N_RING - 1,)),
            pltpu.SemaphoreType.DMA((N_RING - 1,)),
            pltpu.SemaphoreType.DMA((N_RING - 1,)),
        ],
        compiler_params=pltpu.CompilerParams(collective_id=0),
    )(q, k, v)
    return out.transpose(1, 0, 2)[None]
